# Initial kernel scaffold; baseline (speedup 1.0000x reference)
import jax
import jax.numpy as jnp
from jax.experimental import pallas as pl


def kernel(x, edge_index, edge_attr, node_W, node_b, node_ln_g, node_ln_b, self_loop_attr, edge_W, edge_b, edge_ln_g, edge_ln_b, mlp_W1, mlp_b1, mlp_W2, mlp_b2, bn_g, bn_b):
    raise NotImplementedError("write your pallas kernel here")


# speedup vs baseline: 4.4944x; 4.4944x over previous
import functools

import jax
import jax.numpy as jnp
from jax import lax
from jax.experimental import pallas as pl
from jax.experimental.pallas import tpu as pltpu
from jax.experimental.pallas import tpu_sc as plsc

N = 10000
E = 320000
D_NODE = 128
D_EDGE = 16
H = 128
L = 3

CHUNK = 128
NUM_CHUNKS = E // CHUNK
NW = 32
MAX_TRIPS = (NUM_CHUNKS + NW - 1) // NW
ZR = 80
NZB = N // ZR
ZTRIPS = (NZB + 15) // 16


def _ln(v, g, b):
    m = jnp.mean(v, axis=-1, keepdims=True)
    var = jnp.mean((v - m) ** 2, axis=-1, keepdims=True)
    return (v - m) / jnp.sqrt(var + 1e-5) * g + b


def _node_embed_body(x_ref, w_ref, b_ref, g_ref, bb_ref, o_ref):
    h = jnp.dot(x_ref[...], w_ref[...], preferred_element_type=jnp.float32)
    h = _ln(h + b_ref[...], g_ref[...], bb_ref[...])
    o_ref[...] = jnp.maximum(h, 0.0)


def _node_embed(x, w, b, g, bb):
    return pl.pallas_call(
        _node_embed_body,
        out_shape=jax.ShapeDtypeStruct((N, H), jnp.float32),
    )(x, w, b.reshape(1, H), g.reshape(1, H), bb.reshape(1, H))


_EB = 6400


def _edge_embed_body(ea_ref, w_ref, b_ref, g_ref, bb_ref, o_ref):
    h = jnp.dot(ea_ref[...], w_ref[...], preferred_element_type=jnp.float32)
    h = _ln(h + b_ref[...], g_ref[...], bb_ref[...])
    o_ref[...] = jnp.maximum(h, 0.0)


def _edge_embed(edge_attr, w, b, g, bb):
    nblk = E // _EB
    return pl.pallas_call(
        _edge_embed_body,
        grid=(nblk,),
        in_specs=[
            pl.BlockSpec((_EB, D_EDGE), lambda i: (i, 0)),
            pl.BlockSpec((D_EDGE, H), lambda i: (0, 0)),
            pl.BlockSpec((1, H), lambda i: (0, 0)),
            pl.BlockSpec((1, H), lambda i: (0, 0)),
            pl.BlockSpec((1, H), lambda i: (0, 0)),
        ],
        out_specs=pl.BlockSpec((_EB, H), lambda i: (i, 0)),
        out_shape=jax.ShapeDtypeStruct((E, H), jnp.float32),
    )(edge_attr, w, b.reshape(1, H), g.reshape(1, H), bb.reshape(1, H))


def _sc_body(h_hbm, ee_hbm, src_hbm, dst_hbm, z_hbm, out_hbm,
             src_v, dst_v, rows_v, ee_v, aggr_sh, sem):
    cid = lax.axis_index("c")
    sid = lax.axis_index("s")
    wid = sid * 2 + cid

    pltpu.sync_copy(z_hbm, rows_v)

    def zbody(i, carry):
        b = sid + i * 16

        @pl.when(b < NZB)
        def _():
            row0 = pl.multiple_of(b * ZR, ZR)
            pltpu.sync_copy(rows_v.at[pl.ds(0, ZR)], aggr_sh.at[pl.ds(row0, ZR)])

        return carry

    lax.fori_loop(0, ZTRIPS, zbody, 0)
    plsc.subcore_barrier()

    def trip(i, _):
        c = wid + i * NW

        @pl.when(c < NUM_CHUNKS)
        def _():
            base = pl.multiple_of(c * CHUNK, CHUNK)
            pltpu.sync_copy(src_hbm.at[pl.ds(base, CHUNK)], src_v)
            pltpu.sync_copy(dst_hbm.at[pl.ds(base, CHUNK)], dst_v)
            pltpu.async_copy(h_hbm.at[src_v], rows_v, sem).wait()
            pltpu.sync_copy(ee_hbm.at[pl.ds(base, CHUNK)], ee_v)

            def addrow(r, carry):
                for j in range(H // 16):
                    sl = pl.ds(j * 16, 16)
                    rows_v[r, sl] = rows_v[r, sl] + ee_v[r, sl]
                return carry

            lax.fori_loop(0, CHUNK, addrow, 0)
            pltpu.sync_copy(rows_v, aggr_sh.at[dst_v], add=True)

        return 0

    lax.fori_loop(0, MAX_TRIPS, trip, 0)
    plsc.subcore_barrier()

    def dbody(i, carry):
        b = sid + i * 16

        @pl.when(b < NZB)
        def _():
            row0 = pl.multiple_of(b * ZR, ZR)
            pltpu.sync_copy(aggr_sh.at[pl.ds(row0, ZR)], rows_v.at[pl.ds(0, ZR)])
            pltpu.sync_copy(rows_v.at[pl.ds(0, ZR)],
                            out_hbm.at[cid, pl.ds(row0, ZR)])

        return carry

    lax.fori_loop(0, ZTRIPS, dbody, 0)


def _sc_aggregate(h, ee, src, dst, zeros_blk):
    mesh = plsc.VectorSubcoreMesh(core_axis_name="c", subcore_axis_name="s")
    f = functools.partial(
        pl.kernel,
        out_type=jax.ShapeDtypeStruct((2, N, H), jnp.float32),
        mesh=mesh,
        scratch_types=[
            pltpu.VMEM((CHUNK,), jnp.int32),
            pltpu.VMEM((CHUNK,), jnp.int32),
            pltpu.VMEM((CHUNK, H), jnp.float32),
            pltpu.VMEM((CHUNK, H), jnp.float32),
            pltpu.VMEM_SHARED((N, H), jnp.float32),
            pltpu.SemaphoreType.DMA,
        ],
    )(_sc_body)
    return f(h, ee, src, dst, zeros_blk)


def _loop_row_body(sl_ref, ew_ref, eb_ref, eg_ref, ebb_ref, o_ref):
    row = jnp.dot(sl_ref[...], ew_ref[...], preferred_element_type=jnp.float32)
    o_ref[...] = jnp.maximum(_ln(row + eb_ref[...], eg_ref[...], ebb_ref[...]),
                             0.0)


def _loop_row(sl_attr, ew, eb, eg, ebb):
    return pl.pallas_call(
        _loop_row_body,
        out_shape=jax.ShapeDtypeStruct((1, H), jnp.float32),
    )(sl_attr.reshape(1, D_EDGE), ew, eb.reshape(1, H), eg.reshape(1, H),
      ebb.reshape(1, H))


def _mlp_body(p_ref, h_ref, lr_ref, w1_ref, b1_ref, w2_ref, b2_ref,
              bg_ref, bb_ref, o_ref, *, last):
    aggr = p_ref[0] + p_ref[1] + h_ref[...] + lr_ref[...]
    t = jnp.dot(aggr, w1_ref[...], preferred_element_type=jnp.float32)
    t = jax.nn.gelu(t + b1_ref[...])
    o = jnp.dot(t, w2_ref[...], preferred_element_type=jnp.float32)
    o = o + b2_ref[...]
    mu = jnp.mean(o, axis=0, keepdims=True)
    var = jnp.mean((o - mu) ** 2, axis=0, keepdims=True)
    o = (o - mu) / jnp.sqrt(var + 1e-5) * bg_ref[...] + bb_ref[...]
    if not last:
        o = jnp.maximum(o, 0.0)
    o_ref[...] = o


def _mlp_bn(p, h, loop_row, w1, b1, w2, b2, bg, bb, last):
    return pl.pallas_call(
        functools.partial(_mlp_body, last=last),
        out_shape=jax.ShapeDtypeStruct((N, H), jnp.float32),
    )(p, h, loop_row, w1, b1.reshape(1, 2 * H),
      w2, b2.reshape(1, H), bg.reshape(1, H), bb.reshape(1, H))


def kernel(x, edge_index, edge_attr, node_W, node_b, node_ln_g, node_ln_b,
           self_loop_attr, edge_W, edge_b, edge_ln_g, edge_ln_b,
           mlp_W1, mlp_b1, mlp_W2, mlp_b2, bn_g, bn_b):
    src = edge_index[0]
    dst = edge_index[1]
    zeros_blk = jnp.zeros((CHUNK, H), jnp.float32)

    h = _node_embed(x, node_W, node_b, node_ln_g, node_ln_b)
    for l in range(L):
        ee = _edge_embed(edge_attr, edge_W[l], edge_b[l],
                         edge_ln_g[l], edge_ln_b[l])
        p = _sc_aggregate(h, ee, src, dst, zeros_blk)
        lr = _loop_row(self_loop_attr[l], edge_W[l], edge_b[l],
                       edge_ln_g[l], edge_ln_b[l])
        h = _mlp_bn(p, h, lr, mlp_W1[l], mlp_b1[l],
                    mlp_W2[l], mlp_b2[l], bn_g[l], bn_b[l], last=(l == L - 1))
    return h
